# Initial kernel scaffold; baseline (speedup 1.0000x reference)
import jax
import jax.numpy as jnp
from jax.experimental import pallas as pl


def kernel(x, gate_w, gate_b, w1, b1, w2, b2):
    raise NotImplementedError("write your pallas kernel here")


# speedup vs baseline: 5.8180x; 5.8180x over previous
import functools

import jax
import jax.numpy as jnp
from jax import lax
from jax.experimental import pallas as pl
from jax.experimental.pallas import tpu as pltpu
from jax.experimental.pallas import tpu_sc as plsc

N, D, H, E = 2048, 768, 768, 64
TILE = 128
NPAD = N + 7 * E + TILE
NC, NS = 2, 16
NW = NC * NS
RPW = N // NW


def _routing_body(x_ref, gw_ref, gb_ref, slot_ref, offs8_ref, cnts_ref):
    x = x_ref[...]
    logits = jnp.dot(x, gw_ref[...], preferred_element_type=jnp.float32)
    logits = logits + gb_ref[...]
    m = jnp.max(logits, axis=1, keepdims=True)
    lane = lax.broadcasted_iota(jnp.int32, (N, E), 1)
    expert = jnp.min(jnp.where(logits == m, lane, E), axis=1, keepdims=True)
    onehot = (lane == expert).astype(jnp.float32)

    incl = onehot
    k = 1
    while k < N:
        incl = incl + jnp.concatenate(
            [jnp.zeros((k, E), jnp.float32), incl[: N - k]], axis=0)
        k *= 2
    excl = incl - onehot

    counts = jnp.sum(onehot, axis=0, keepdims=True)
    counts8 = jnp.floor((counts + 7.0) / 8.0) * 8.0
    r = lax.broadcasted_iota(jnp.int32, (E, E), 0)
    c = lax.broadcasted_iota(jnp.int32, (E, E), 1)
    tri = (r < c).astype(jnp.float32)
    offs = jnp.dot(counts8, tri, preferred_element_type=jnp.float32)

    slot = jnp.sum(onehot * (excl + offs), axis=1, keepdims=True)
    slot_ref[...] = slot.astype(jnp.int32)
    offs8_ref[...] = (offs / 8.0).astype(jnp.int32)
    cnts_ref[...] = counts.astype(jnp.int32)


def _routing(x, gate_w, gate_b):
    return pl.pallas_call(
        _routing_body,
        out_shape=(
            jax.ShapeDtypeStruct((N, 1), jnp.int32),
            jax.ShapeDtypeStruct((1, E), jnp.int32),
            jax.ShapeDtypeStruct((1, E), jnp.int32),
        ),
    )(x, gate_w, gate_b.reshape(1, E))


def _wid():
    return lax.axis_index("s") * NC + lax.axis_index("c")


def _scatter_body(x_hbm, slot_hbm, xs_hbm, idx_v, rows_v, sem):
    base = _wid() * RPW
    pltpu.sync_copy(slot_hbm.at[pl.ds(base, RPW)], idx_v)
    pltpu.sync_copy(x_hbm.at[pl.ds(base, RPW)], rows_v)
    pltpu.async_copy(rows_v, xs_hbm.at[idx_v], sem).wait()


def _gather_body(ys_hbm, slot_hbm, out_hbm, idx_v, rows_v, sem):
    base = _wid() * RPW
    pltpu.sync_copy(slot_hbm.at[pl.ds(base, RPW)], idx_v)
    pltpu.async_copy(ys_hbm.at[idx_v], rows_v, sem).wait()
    pltpu.sync_copy(rows_v, out_hbm.at[pl.ds(base, RPW)])


@functools.cache
def _sc_kernels():
    mesh = plsc.VectorSubcoreMesh(core_axis_name="c", subcore_axis_name="s")
    scratch = [
        pltpu.VMEM((RPW,), jnp.int32),
        pltpu.VMEM((RPW, D), jnp.float32),
        pltpu.SemaphoreType.DMA,
    ]
    scatter = pl.kernel(
        _scatter_body, mesh=mesh,
        out_type=jax.ShapeDtypeStruct((NPAD, D), jnp.float32),
        scratch_types=scratch,
    )
    gather = pl.kernel(
        _gather_body, mesh=mesh,
        out_type=jax.ShapeDtypeStruct((N, D), jnp.float32),
        scratch_types=scratch,
    )
    return scatter, gather


def _expert_body(offs8_ref, cnts_ref, xs_ref, w1_ref, b1_ref, w2_ref, b2_ref,
                 ys_ref):
    e = pl.program_id(0)
    nch = (cnts_ref[e] + TILE - 1) // TILE
    w1 = w1_ref[0]
    w2 = w2_ref[0]
    b1 = b1_ref[0]
    b2 = b2_ref[0]

    def body(k, carry):
        s = (offs8_ref[e] + k * (TILE // 8)) * 8
        xb = xs_ref[pl.ds(s, TILE), :]
        h = jnp.maximum(
            jnp.dot(xb, w1, preferred_element_type=jnp.float32) + b1, 0.0)
        y = jnp.dot(h, w2, preferred_element_type=jnp.float32) + b2
        ys_ref[pl.ds(s, TILE), :] = y
        return carry

    lax.fori_loop(0, nch, body, 0)


def _expert_ffn(offs8, cnts, xs, w1, b1, w2, b2):
    grid_spec = pltpu.PrefetchScalarGridSpec(
        num_scalar_prefetch=2,
        grid=(E,),
        in_specs=[
            pl.BlockSpec((NPAD, D), lambda e, o, c: (0, 0)),
            pl.BlockSpec((1, D, H), lambda e, o, c: (e, 0, 0)),
            pl.BlockSpec((1, 1, H), lambda e, o, c: (e, 0, 0)),
            pl.BlockSpec((1, H, D), lambda e, o, c: (e, 0, 0)),
            pl.BlockSpec((1, 1, D), lambda e, o, c: (e, 0, 0)),
        ],
        out_specs=pl.BlockSpec((NPAD, D), lambda e, o, c: (0, 0)),
    )
    return pl.pallas_call(
        _expert_body,
        grid_spec=grid_spec,
        out_shape=jax.ShapeDtypeStruct((NPAD, D), jnp.float32),
        compiler_params=pltpu.CompilerParams(
            dimension_semantics=("arbitrary",)),
    )(offs8, cnts, xs, w1, b1.reshape(E, 1, H), w2, b2.reshape(E, 1, D))


def kernel(x, gate_w, gate_b, w1, b1, w2, b2):
    slot2d, offs8_2d, cnts2d = _routing(x, gate_w, gate_b)
    slot = slot2d.reshape(N)
    offs8 = offs8_2d.reshape(E)
    cnts = cnts2d.reshape(E)
    sc_scatter, sc_gather = _sc_kernels()
    xs = sc_scatter(x, slot)
    ys = _expert_ffn(offs8, cnts, xs, w1, b1, w2, b2)
    return sc_gather(ys, slot)


# speedup vs baseline: 5.8760x; 1.0100x over previous
import functools

import jax
import jax.numpy as jnp
from jax import lax
from jax.experimental import pallas as pl
from jax.experimental.pallas import tpu as pltpu
from jax.experimental.pallas import tpu_sc as plsc

N, D, H, E = 2048, 768, 768, 64
TILE = 64
NPAD = N + 7 * E + TILE
NC, NS = 2, 16
NW = NC * NS
RPW = N // NW


def _routing_body(x_ref, gw_ref, gb_ref, slot_ref, offs8_ref, cnts_ref):
    x = x_ref[...]
    logits = jnp.dot(x, gw_ref[...], preferred_element_type=jnp.float32)
    logits = logits + gb_ref[...]
    m = jnp.max(logits, axis=1, keepdims=True)
    lane = lax.broadcasted_iota(jnp.int32, (N, E), 1)
    expert = jnp.min(jnp.where(logits == m, lane, E), axis=1, keepdims=True)
    onehot = (lane == expert).astype(jnp.float32)

    incl = onehot
    k = 1
    while k < N:
        incl = incl + jnp.concatenate(
            [jnp.zeros((k, E), jnp.float32), incl[: N - k]], axis=0)
        k *= 2
    excl = incl - onehot

    counts = jnp.sum(onehot, axis=0, keepdims=True)
    counts8 = jnp.floor((counts + 7.0) / 8.0) * 8.0
    r = lax.broadcasted_iota(jnp.int32, (E, E), 0)
    c = lax.broadcasted_iota(jnp.int32, (E, E), 1)
    tri = (r < c).astype(jnp.float32)
    offs = jnp.dot(counts8, tri, preferred_element_type=jnp.float32)

    slot = jnp.sum(onehot * (excl + offs), axis=1, keepdims=True)
    slot_ref[...] = slot.astype(jnp.int32)
    offs8_ref[...] = (offs / 8.0).astype(jnp.int32)
    cnts_ref[...] = counts.astype(jnp.int32)


def _routing(x, gate_w, gate_b):
    return pl.pallas_call(
        _routing_body,
        out_shape=(
            jax.ShapeDtypeStruct((N, 1), jnp.int32),
            jax.ShapeDtypeStruct((1, E), jnp.int32),
            jax.ShapeDtypeStruct((1, E), jnp.int32),
        ),
    )(x, gate_w, gate_b.reshape(1, E))


def _wid():
    return lax.axis_index("s") * NC + lax.axis_index("c")


def _scatter_body(x_hbm, slot_hbm, xs_hbm, idx_v, rows_v, sem):
    base = _wid() * RPW
    pltpu.sync_copy(slot_hbm.at[pl.ds(base, RPW)], idx_v)
    pltpu.sync_copy(x_hbm.at[pl.ds(base, RPW)], rows_v)
    pltpu.async_copy(rows_v, xs_hbm.at[idx_v], sem).wait()


def _gather_body(ys_hbm, slot_hbm, out_hbm, idx_v, rows_v, sem):
    base = _wid() * RPW
    pltpu.sync_copy(slot_hbm.at[pl.ds(base, RPW)], idx_v)
    pltpu.async_copy(ys_hbm.at[idx_v], rows_v, sem).wait()
    pltpu.sync_copy(rows_v, out_hbm.at[pl.ds(base, RPW)])


@functools.cache
def _sc_kernels():
    mesh = plsc.VectorSubcoreMesh(core_axis_name="c", subcore_axis_name="s")
    scratch = [
        pltpu.VMEM((RPW,), jnp.int32),
        pltpu.VMEM((RPW, D), jnp.float32),
        pltpu.SemaphoreType.DMA,
    ]
    scatter = pl.kernel(
        _scatter_body, mesh=mesh,
        out_type=jax.ShapeDtypeStruct((NPAD, D), jnp.float32),
        scratch_types=scratch,
    )
    gather = pl.kernel(
        _gather_body, mesh=mesh,
        out_type=jax.ShapeDtypeStruct((N, D), jnp.float32),
        scratch_types=scratch,
    )
    return scatter, gather


def _expert_body(offs8_ref, cnts_ref, xs_ref, w1_ref, b1_ref, w2_ref, b2_ref,
                 ys_ref):
    e = pl.program_id(0)
    nch = (cnts_ref[e] + TILE - 1) // TILE
    w1 = w1_ref[0]
    w2 = w2_ref[0]
    b1 = b1_ref[0]
    b2 = b2_ref[0]

    def body(k, carry):
        s = (offs8_ref[e] + k * (TILE // 8)) * 8
        xb = xs_ref[pl.ds(s, TILE), :]
        h = jnp.maximum(
            jnp.dot(xb, w1, preferred_element_type=jnp.float32) + b1, 0.0)
        y = jnp.dot(h, w2, preferred_element_type=jnp.float32) + b2
        ys_ref[pl.ds(s, TILE), :] = y
        return carry

    lax.fori_loop(0, nch, body, 0)


def _expert_ffn(offs8, cnts, xs, w1, b1, w2, b2):
    grid_spec = pltpu.PrefetchScalarGridSpec(
        num_scalar_prefetch=2,
        grid=(E,),
        in_specs=[
            pl.BlockSpec((NPAD, D), lambda e, o, c: (0, 0)),
            pl.BlockSpec((1, D, H), lambda e, o, c: (e, 0, 0)),
            pl.BlockSpec((1, 1, H), lambda e, o, c: (e, 0, 0)),
            pl.BlockSpec((1, H, D), lambda e, o, c: (e, 0, 0)),
            pl.BlockSpec((1, 1, D), lambda e, o, c: (e, 0, 0)),
        ],
        out_specs=pl.BlockSpec((NPAD, D), lambda e, o, c: (0, 0)),
    )
    return pl.pallas_call(
        _expert_body,
        grid_spec=grid_spec,
        out_shape=jax.ShapeDtypeStruct((NPAD, D), jnp.float32),
        compiler_params=pltpu.CompilerParams(
            dimension_semantics=("arbitrary",)),
    )(offs8, cnts, xs, w1, b1.reshape(E, 1, H), w2, b2.reshape(E, 1, D))


def kernel(x, gate_w, gate_b, w1, b1, w2, b2):
    slot2d, offs8_2d, cnts2d = _routing(x, gate_w, gate_b)
    slot = slot2d.reshape(N)
    offs8 = offs8_2d.reshape(E)
    cnts = cnts2d.reshape(E)
    sc_scatter, sc_gather = _sc_kernels()
    xs = sc_scatter(x, slot)
    ys = _expert_ffn(offs8, cnts, xs, w1, b1, w2, b2)
    return sc_gather(ys, slot)


# speedup vs baseline: 5.9327x; 1.0097x over previous
import functools

import jax
import jax.numpy as jnp
from jax import lax
from jax.experimental import pallas as pl
from jax.experimental.pallas import tpu as pltpu
from jax.experimental.pallas import tpu_sc as plsc

N, D, H, E = 2048, 768, 768, 64
TILE = 64
NPAD = N + 7 * E + TILE
NC, NS = 2, 16
NW = NC * NS
RPW = N // NW


def _routing_body(x_ref, gw_ref, gb_ref, slot_ref, offs8_ref, cnts_ref):
    x = x_ref[...]
    logits = jnp.dot(x, gw_ref[...], preferred_element_type=jnp.float32)
    logits = logits + gb_ref[...]
    m = jnp.max(logits, axis=1, keepdims=True)
    lane = lax.broadcasted_iota(jnp.int32, (N, E), 1)
    expert = jnp.min(jnp.where(logits == m, lane, E), axis=1, keepdims=True)
    onehot = (lane == expert).astype(jnp.float32)

    incl = onehot
    k = 1
    while k < N:
        incl = incl + jnp.concatenate(
            [jnp.zeros((k, E), jnp.float32), incl[: N - k]], axis=0)
        k *= 2
    excl = incl - onehot

    counts = jnp.sum(onehot, axis=0, keepdims=True)
    counts8 = jnp.floor((counts + 7.0) / 8.0) * 8.0
    r = lax.broadcasted_iota(jnp.int32, (E, E), 0)
    c = lax.broadcasted_iota(jnp.int32, (E, E), 1)
    tri = (r < c).astype(jnp.float32)
    offs = jnp.dot(counts8, tri, preferred_element_type=jnp.float32)

    slot = jnp.sum(onehot * (excl + offs), axis=1, keepdims=True)
    slot_ref[...] = slot.astype(jnp.int32)
    offs8_ref[...] = (offs / 8.0).astype(jnp.int32)
    cnts_ref[...] = counts.astype(jnp.int32)


def _routing(x, gate_w, gate_b):
    return pl.pallas_call(
        _routing_body,
        out_shape=(
            jax.ShapeDtypeStruct((N, 1), jnp.int32),
            jax.ShapeDtypeStruct((1, E), jnp.int32),
            jax.ShapeDtypeStruct((1, E), jnp.int32),
        ),
    )(x, gate_w, gate_b.reshape(1, E))


def _wid():
    return lax.axis_index("s") * NC + lax.axis_index("c")


def _scatter_body(x_hbm, slot_hbm, xs_hbm, idx_v, rows_v, sem):
    base = _wid() * RPW
    pltpu.sync_copy(slot_hbm.at[pl.ds(base, RPW)], idx_v)
    pltpu.sync_copy(x_hbm.at[pl.ds(base, RPW)], rows_v)
    pltpu.async_copy(rows_v, xs_hbm.at[idx_v], sem).wait()


def _gather_body(ys_hbm, slot_hbm, out_hbm, idx_v, rows_v, sem):
    base = _wid() * RPW
    pltpu.sync_copy(slot_hbm.at[pl.ds(base, RPW)], idx_v)
    pltpu.async_copy(ys_hbm.at[idx_v], rows_v, sem).wait()
    pltpu.sync_copy(rows_v, out_hbm.at[pl.ds(base, RPW)])


@functools.cache
def _sc_kernels():
    mesh = plsc.VectorSubcoreMesh(core_axis_name="c", subcore_axis_name="s")
    scratch = [
        pltpu.VMEM((RPW,), jnp.int32),
        pltpu.VMEM((RPW, D), jnp.float32),
        pltpu.SemaphoreType.DMA,
    ]
    scatter = pl.kernel(
        _scatter_body, mesh=mesh,
        out_type=jax.ShapeDtypeStruct((NPAD, D), jnp.float32),
        scratch_types=scratch,
    )
    gather = pl.kernel(
        _gather_body, mesh=mesh,
        out_type=jax.ShapeDtypeStruct((N, D), jnp.float32),
        scratch_types=scratch,
    )
    return scatter, gather


H2 = H // 2
D2 = D // 2


def _expert_body(offs8_ref, cnts_ref, xs_ref, w1a_ref, w1b_ref, b1_ref,
                 w2a_ref, w2b_ref, b2_ref, ys_ref):
    e = pl.program_id(0)
    nch = (cnts_ref[e] + TILE - 1) // TILE
    w1a = w1a_ref[0]
    w1b = w1b_ref[0]
    w2a = w2a_ref[0]
    w2b = w2b_ref[0]
    b1 = b1_ref[0]
    b2 = b2_ref[0]

    def body(k, carry):
        s = (offs8_ref[e] + k * (TILE // 8)) * 8
        xb = xs_ref[pl.ds(s, TILE), :]
        h1 = jnp.maximum(
            jnp.dot(xb, w1a, preferred_element_type=jnp.float32)
            + b1[:, :H2], 0.0)
        h2 = jnp.maximum(
            jnp.dot(xb, w1b, preferred_element_type=jnp.float32)
            + b1[:, H2:], 0.0)
        h = jnp.concatenate([h1, h2], axis=1)
        y1 = jnp.dot(h, w2a, preferred_element_type=jnp.float32)
        y2 = jnp.dot(h, w2b, preferred_element_type=jnp.float32)
        ys_ref[pl.ds(s, TILE), :] = jnp.concatenate([y1, y2], axis=1) + b2
        return carry

    lax.fori_loop(0, nch, body, 0)


def _expert_ffn(offs8, cnts, xs, w1, b1, w2, b2):
    grid_spec = pltpu.PrefetchScalarGridSpec(
        num_scalar_prefetch=2,
        grid=(E,),
        in_specs=[
            pl.BlockSpec((NPAD, D), lambda e, o, c: (0, 0)),
            pl.BlockSpec((1, D, H2), lambda e, o, c: (e, 0, 0)),
            pl.BlockSpec((1, D, H2), lambda e, o, c: (e, 0, 1)),
            pl.BlockSpec((1, 1, H), lambda e, o, c: (e, 0, 0)),
            pl.BlockSpec((1, H, D2), lambda e, o, c: (e, 0, 0)),
            pl.BlockSpec((1, H, D2), lambda e, o, c: (e, 0, 1)),
            pl.BlockSpec((1, 1, D), lambda e, o, c: (e, 0, 0)),
        ],
        out_specs=pl.BlockSpec((NPAD, D), lambda e, o, c: (0, 0)),
    )
    return pl.pallas_call(
        _expert_body,
        grid_spec=grid_spec,
        out_shape=jax.ShapeDtypeStruct((NPAD, D), jnp.float32),
        compiler_params=pltpu.CompilerParams(
            dimension_semantics=("arbitrary",)),
    )(offs8, cnts, xs, w1, w1, b1.reshape(E, 1, H), w2, w2,
      b2.reshape(E, 1, D))


def kernel(x, gate_w, gate_b, w1, b1, w2, b2):
    slot2d, offs8_2d, cnts2d = _routing(x, gate_w, gate_b)
    slot = slot2d.reshape(N)
    offs8 = offs8_2d.reshape(E)
    cnts = cnts2d.reshape(E)
    sc_scatter, sc_gather = _sc_kernels()
    xs = sc_scatter(x, slot)
    ys = _expert_ffn(offs8, cnts, xs, w1, b1, w2, b2)
    return sc_gather(ys, slot)
